# baseline (speedup 1.0000x reference)
import jax
import jax.numpy as jnp
from jax import lax
from jax.experimental import pallas as pl
from jax.experimental.pallas import tpu as pltpu
from jax.experimental.pallas import tpu_sc as plsc

NUM_CORES = 2
NUM_SUBCORES = 16
LANES = 16
NW = NUM_CORES * NUM_SUBCORES
BATCH = 16384
EMBED_DIM = 32
BIAS_COL = EMBED_DIM
ROW_W = EMBED_DIM + 1
N_PER_W = BATCH // NW
PASS_ROWS = 128
N_PASS = N_PER_W // PASS_ROWS
PASS_GROUPS = PASS_ROWS // LANES


def _mf_kernel(user_hbm, item_hbm, user_e_hbm, item_e_hbm, user_b_hbm,
               item_b_hbm, out_hbm, u_idx, i_idx, u_rows, i_rows, out_v,
               sem):
    wid = lax.axis_index("s") * NUM_CORES + lax.axis_index("c")
    base = wid * N_PER_W

    pltpu.sync_copy(user_hbm.at[pl.ds(base, N_PER_W)], u_idx)
    pltpu.sync_copy(item_hbm.at[pl.ds(base, N_PER_W)], i_idx)

    iota16 = lax.iota(jnp.int32, LANES)
    biascol = jnp.full((LANES,), BIAS_COL, dtype=jnp.int32)

    LOOKAHEAD = 2

    def pass_body(p, carry):
        groups = {}

        def fire(g):
            gsem = sem.at[g]
            k0 = p * PASS_ROWS + g * LANES
            vu = u_idx[pl.ds(k0, LANES)]
            vi = i_idx[pl.ds(k0, LANES)]
            cps = []
            for l in range(LANES):
                k = g * LANES + l
                cps.append(pltpu.async_copy(
                    user_e_hbm.at[vu[l]],
                    u_rows.at[k, pl.ds(0, EMBED_DIM)], gsem))
                cps.append(pltpu.async_copy(
                    item_e_hbm.at[vi[l]],
                    i_rows.at[k, pl.ds(0, EMBED_DIM)], gsem))
                cps.append(pltpu.async_copy(
                    user_b_hbm.at[vu[l]],
                    u_rows.at[k, pl.ds(BIAS_COL, 1)], gsem))
                cps.append(pltpu.async_copy(
                    item_b_hbm.at[vi[l]],
                    i_rows.at[k, pl.ds(BIAS_COL, 1)], gsem))
            groups[g] = cps

        for g in range(LOOKAHEAD):
            fire(g)
        for g in range(PASS_GROUPS):
            for c in groups.pop(g):
                c.wait()
            row0 = g * LANES
            rows = row0 + iota16
            acc = (plsc.load_gather(u_rows, [rows, biascol])
                   + plsc.load_gather(i_rows, [rows, biascol]))
            for d in range(EMBED_DIM):
                cold = jnp.full((LANES,), d, dtype=jnp.int32)
                u = plsc.load_gather(u_rows, [rows, cold])
                v = plsc.load_gather(i_rows, [rows, cold])
                acc = acc + u * v
            out_v[pl.ds(p * PASS_ROWS + row0, LANES)] = acc
            if g + LOOKAHEAD < PASS_GROUPS:
                fire(g + LOOKAHEAD)
        return carry

    lax.fori_loop(0, N_PASS, pass_body, 0)

    pltpu.sync_copy(out_v, out_hbm.at[pl.ds(base, N_PER_W)])


@jax.jit
def kernel(user, item, user_e, item_e, user_b, item_b):
    mesh = plsc.VectorSubcoreMesh(core_axis_name="c", subcore_axis_name="s")
    run = pl.kernel(
        _mf_kernel,
        out_type=jax.ShapeDtypeStruct((BATCH,), jnp.float32),
        mesh=mesh,
        scratch_types=[
            pltpu.VMEM((N_PER_W,), jnp.int32),
            pltpu.VMEM((N_PER_W,), jnp.int32),
            pltpu.VMEM((PASS_ROWS, ROW_W), jnp.float32),
            pltpu.VMEM((PASS_ROWS, ROW_W), jnp.float32),
            pltpu.VMEM((N_PER_W,), jnp.float32),
            pltpu.SemaphoreType.DMA((PASS_GROUPS,)),
        ],
        compiler_params=pltpu.CompilerParams(
            needs_layout_passes=False, use_tc_tiling_on_sc=True),
    )
    return run(user.astype(jnp.int32), item.astype(jnp.int32),
               user_e, item_e, user_b, item_b)


# speedup vs baseline: 1.1675x; 1.1675x over previous
import jax
import jax.numpy as jnp
from jax import lax
from jax.experimental import pallas as pl
from jax.experimental.pallas import tpu as pltpu
from jax.experimental.pallas import tpu_sc as plsc

NUM_CORES = 2
NUM_SUBCORES = 16
LANES = 16
NW = NUM_CORES * NUM_SUBCORES
BATCH = 16384
EMBED_DIM = 32
ROWS_PER_SLAB = 4
SLAB_W = ROWS_PER_SLAB * EMBED_DIM
N_PER_W = BATCH // NW
CHUNK = 128
N_CHUNKS = N_PER_W // CHUNK
CHUNK_GROUPS = CHUNK // LANES
NRING = 2


def _mf_kernel(user_hbm, item_hbm, ue_hbm, ie_hbm, ub_hbm, ib_hbm, out_hbm,
               u_idx, i_idx, u_q, i_q, u_slab, i_slab, u_bias, i_bias,
               out_v, sems):
    wid = lax.axis_index("s") * NUM_CORES + lax.axis_index("c")
    base = wid * N_PER_W

    pltpu.sync_copy(user_hbm.at[pl.ds(base, N_PER_W)], u_idx)
    pltpu.sync_copy(item_hbm.at[pl.ds(base, N_PER_W)], i_idx)

    for v0 in range(0, N_PER_W, LANES):
        u_q[pl.ds(v0, LANES)] = jax.lax.shift_right_logical(
            u_idx[pl.ds(v0, LANES)], 2)
        i_q[pl.ds(v0, LANES)] = jax.lax.shift_right_logical(
            i_idx[pl.ds(v0, LANES)], 2)

    def fire(c):
        ring = c % NRING
        sem = sems.at[ring]
        sl = pl.ds(c * CHUNK, CHUNK)
        cps = [
            pltpu.async_copy(ue_hbm.at[u_q.at[sl]], u_slab.at[ring], sem),
            pltpu.async_copy(ie_hbm.at[i_q.at[sl]], i_slab.at[ring], sem),
            pltpu.async_copy(ub_hbm.at[u_idx.at[sl]],
                             u_bias.at[pl.ds(c * CHUNK, CHUNK)], sem),
            pltpu.async_copy(ib_hbm.at[i_idx.at[sl]],
                             i_bias.at[pl.ds(c * CHUNK, CHUNK)], sem),
        ]
        return cps

    iota16 = lax.iota(jnp.int32, LANES)

    def compute(c):
        ring = c % NRING

        for g in range(CHUNK_GROUPS):
            k0 = c * CHUNK + g * LANES
            slot = g * LANES + iota16
            vu = u_idx[pl.ds(k0, LANES)]
            vi = i_idx[pl.ds(k0, LANES)]
            uc0 = jax.lax.shift_left(jnp.bitwise_and(vu, 3), 5)
            ic0 = jax.lax.shift_left(jnp.bitwise_and(vi, 3), 5)
            acc = u_bias[pl.ds(k0, LANES)] + i_bias[pl.ds(k0, LANES)]
            for d in range(EMBED_DIM):
                u = plsc.load_gather(u_slab, [jnp.full((LANES,), ring,
                                                       jnp.int32),
                                              slot, uc0 + d])
                v = plsc.load_gather(i_slab, [jnp.full((LANES,), ring,
                                                       jnp.int32),
                                              slot, ic0 + d])
                acc = acc + u * v
            out_v[pl.ds(k0, LANES)] = acc

    pending = {}
    for c in range(NRING):
        pending[c] = fire(c)
    for c in range(N_CHUNKS):
        for cp in pending.pop(c):
            cp.wait()
        compute(c)
        if c + NRING < N_CHUNKS:
            pending[c + NRING] = fire(c + NRING)

    pltpu.sync_copy(out_v, out_hbm.at[pl.ds(base, N_PER_W)])


@jax.jit
def kernel(user, item, user_e, item_e, user_b, item_b):
    ue2 = user_e.reshape(user_e.shape[0] // ROWS_PER_SLAB, SLAB_W)
    ie2 = item_e.reshape(item_e.shape[0] // ROWS_PER_SLAB, SLAB_W)
    ub1 = user_b.reshape(-1)
    ib1 = item_b.reshape(-1)

    mesh = plsc.VectorSubcoreMesh(core_axis_name="c", subcore_axis_name="s")
    run = pl.kernel(
        _mf_kernel,
        out_type=jax.ShapeDtypeStruct((BATCH,), jnp.float32),
        mesh=mesh,
        scratch_types=[
            pltpu.VMEM((N_PER_W,), jnp.int32),
            pltpu.VMEM((N_PER_W,), jnp.int32),
            pltpu.VMEM((N_PER_W,), jnp.int32),
            pltpu.VMEM((N_PER_W,), jnp.int32),
            pltpu.VMEM((NRING, CHUNK, SLAB_W), jnp.float32),
            pltpu.VMEM((NRING, CHUNK, SLAB_W), jnp.float32),
            pltpu.VMEM((N_PER_W,), jnp.float32),
            pltpu.VMEM((N_PER_W,), jnp.float32),
            pltpu.VMEM((N_PER_W,), jnp.float32),
            pltpu.SemaphoreType.DMA((NRING,)),
        ],
        compiler_params=pltpu.CompilerParams(
            needs_layout_passes=False, use_tc_tiling_on_sc=False),
    )
    return run(user.astype(jnp.int32), item.astype(jnp.int32),
               ue2, ie2, ub1, ib1)
